# baseline (speedup 1.0000x reference)
import functools

import jax
import jax.numpy as jnp
from jax import lax
from jax.experimental import pallas as pl
from jax.experimental.pallas import tpu as pltpu
from jax.experimental.pallas import tpu_sc as plsc

NS = 16
CH = 128
V = 100001
D = 32
V_AL = 99968
VP = 100096
V4 = VP // 4
N_CHUNK = V_AL // CH
CPT = -(-N_CHUNK // NS)


def _tail_packed(table):
    tail = jnp.pad(table[V_AL:], ((0, 128 - (V - V_AL)), (0, 0)))
    return tail.reshape(D, 4 * D)


def _sc_gather(umod, dmod, ut_t, ot_t, utail, otail):
    B = umod.shape[0]
    b_per_tile = B // NS
    n_ch = b_per_tile // CH

    uid3 = umod.reshape(NS, n_ch, CH)
    did3 = dmod.reshape(NS, n_ch, CH)

    mesh = plsc.VectorSubcoreMesh(core_axis_name="c", subcore_axis_name="s")

    @functools.partial(
        pl.kernel,
        out_type=(jax.ShapeDtypeStruct((B, CH), jnp.float32),
                  jax.ShapeDtypeStruct((B, CH), jnp.float32),
                  jax.ShapeDtypeStruct((V4, CH), jnp.float32),
                  jax.ShapeDtypeStruct((V4, CH), jnp.float32)),
        mesh=mesh,
        scratch_types=[
            pltpu.VMEM((D, CH), jnp.float32),
            pltpu.VMEM((D, CH), jnp.float32),
            pltpu.VMEM((D, CH), jnp.float32),
            pltpu.VMEM((D, CH), jnp.float32),
            pltpu.VMEM((n_ch, CH), jnp.int32),
            pltpu.VMEM((CH, CH), jnp.float32),
            pltpu.VMEM((CH, CH), jnp.float32),
            pltpu.SemaphoreType.DMA,
            pltpu.SemaphoreType.DMA,
            pltpu.SemaphoreType.DMA,
            pltpu.SemaphoreType.DMA,
        ],
        compiler_params=pltpu.CompilerParams(use_tc_tiling_on_sc=False, needs_layout_passes=False),
    )
    def gather_kernel(uid_hbm, did_hbm, utt_hbm, ott_hbm,
                      utail_hbm, otail_hbm,
                      uout_hbm, oout_hbm, upk_hbm, opk_hbm,
                      in0, in1, tr0, tr1, idxv, gbuf0, gbuf1,
                      semi0, semi1, semo, semg):
        c = lax.axis_index("c")
        s = lax.axis_index("s")
        iota = lax.iota(jnp.int32, 16)

        row_idx = [(16 * h + iota) // 4 for h in range(8)]
        col_pat = [((16 * h + iota) % 4) * D for h in range(8)]

        def transpose_chunk(in_v, tr_v):
            for cp in range(D):
                for h in range(8):
                    v = in_v[cp, pl.ds(16 * h, 16)]
                    plsc.store_scatter(
                        tr_v, [row_idx[h], col_pat[h] + cp], v)

        def work(idx_hbm, tp_ref, tail_ref, pk_ref, out_ref):
            last = N_CHUNK - 1

            def desc(k, buf, sem):
                c0 = pl.multiple_of(jnp.minimum(k * NS + s, last) * CH, CH)
                return pltpu.make_async_copy(
                    tp_ref.at[slice(None), pl.ds(c0, CH)], buf, sem)

            desc(0, in0, semi0).start()
            desc(1, in1, semi1).start()

            @pl.when(s == 0)
            def _():
                pltpu.sync_copy(tail_ref, gbuf0.at[pl.ds(0, D)])
                pltpu.sync_copy(gbuf0.at[pl.ds(0, D)],
                                pk_ref.at[pl.ds(V_AL // 4, D)])

            def step(j, carry):
                for off, in_v, tr_v, semi in ((0, in0, tr0, semi0),
                                              (1, in1, tr1, semi1)):
                    k = 2 * j + off

                    @pl.when(k < CPT)
                    def _():
                        desc(k, in_v, semi).wait()
                        transpose_chunk(in_v, tr_v)

                        @pl.when(k + 2 < CPT)
                        def _():
                            desc(k + 2, in_v, semi).start()

                        r0 = pl.multiple_of(
                            jnp.minimum(k * NS + s, last) * D, D)
                        pltpu.sync_copy(tr_v, pk_ref.at[pl.ds(r0, D)])
                return carry

            lax.fori_loop(0, (CPT + 1) // 2, step, 0)
            plsc.subcore_barrier()

            pltpu.sync_copy(idx_hbm.at[s], idxv)
            base = s * b_per_tile
            gbufs = (gbuf0, gbuf1)
            copies = [None] * n_ch
            for j in range(n_ch):
                copies[j] = pltpu.async_copy(
                    pk_ref.at[idxv.at[j]], gbufs[j % 2], semg)
                if j >= 1:
                    copies[j - 1].wait()
                    pltpu.sync_copy(
                        gbufs[(j - 1) % 2],
                        out_ref.at[pl.ds(pl.multiple_of(base + (j - 1) * CH, CH), CH)])
            copies[n_ch - 1].wait()
            pltpu.sync_copy(
                gbufs[(n_ch - 1) % 2],
                out_ref.at[pl.ds(pl.multiple_of(base + (n_ch - 1) * CH, CH), CH)])

        @pl.when(c == 0)
        def _():
            work(uid_hbm, utt_hbm, utail_hbm, upk_hbm, uout_hbm)

        @pl.when(c == 1)
        def _():
            work(did_hbm, ott_hbm, otail_hbm, opk_hbm, oout_hbm)

    outs = gather_kernel(uid3, did3, ut_t, ot_t, utail, otail)
    return outs[0], outs[1]


def _mlp_body(us_ref, os_ref, uq_ref, oq_ref, w1u_ref, w1o_ref, b1_ref,
              w2_ref, b2_ref, w3t_ref, b3_ref, out_ref):
    u = uq_ref[:, 0:1] * us_ref[:, 0 * D:1 * D]
    o = oq_ref[:, 0:1] * os_ref[:, 0 * D:1 * D]
    for q in range(1, 4):
        u = u + uq_ref[:, q:q + 1] * us_ref[:, q * D:(q + 1) * D]
        o = o + oq_ref[:, q:q + 1] * os_ref[:, q * D:(q + 1) * D]
    h1 = jnp.dot(u, w1u_ref[...], preferred_element_type=jnp.float32)
    h1 = h1 + jnp.dot(o, w1o_ref[...], preferred_element_type=jnp.float32)
    h1 = jnp.maximum(h1 + b1_ref[...], 0.0)
    h2 = jnp.dot(h1, w2_ref[...], preferred_element_type=jnp.float32)
    h2 = jnp.maximum(h2 + b2_ref[...], 0.0)
    out_ref[...] = (jnp.sum(h2 * w3t_ref[...], axis=1, keepdims=True)
                    + b3_ref[...])


def _mlp(u_sup, o_sup, uq, oq, W1, b1, W2, b2, W3, b3, chunk=2048):
    B = u_sup.shape[0]
    H1 = W1.shape[1]
    H2 = W2.shape[1]
    w1u = W1[:D]
    w1o = W1[D:]
    b1r = b1.reshape(1, H1)
    b2r = b2.reshape(1, H2)
    w3t = W3.reshape(1, H2)
    b3r = b3.reshape(1, 1)
    grid = (B // chunk,)
    return pl.pallas_call(
        _mlp_body,
        grid=grid,
        in_specs=[
            pl.BlockSpec((chunk, CH), lambda i: (i, 0)),
            pl.BlockSpec((chunk, CH), lambda i: (i, 0)),
            pl.BlockSpec((chunk, 4), lambda i: (i, 0)),
            pl.BlockSpec((chunk, 4), lambda i: (i, 0)),
            pl.BlockSpec((D, H1), lambda i: (0, 0)),
            pl.BlockSpec((D, H1), lambda i: (0, 0)),
            pl.BlockSpec((1, H1), lambda i: (0, 0)),
            pl.BlockSpec((H1, H2), lambda i: (0, 0)),
            pl.BlockSpec((1, H2), lambda i: (0, 0)),
            pl.BlockSpec((1, H2), lambda i: (0, 0)),
            pl.BlockSpec((1, 1), lambda i: (0, 0)),
        ],
        out_specs=pl.BlockSpec((chunk, 1), lambda i: (i, 0)),
        out_shape=jax.ShapeDtypeStruct((B, 1), jnp.float32),
    )(u_sup, o_sup, uq, oq, w1u, w1o, b1r, W2, b2r, w3t, b3r)


def kernel(user_id, destination, user_table, origin_table,
           W1, b1, W2, b2, W3, b3):
    uid = user_id.astype(jnp.int32)
    did = destination.astype(jnp.int32)
    umod = uid // 4
    dmod = did // 4
    lanes = jnp.arange(4, dtype=jnp.int32)[None, :]
    uq = ((uid % 4)[:, None] == lanes).astype(jnp.float32)
    oq = ((did % 4)[:, None] == lanes).astype(jnp.float32)
    ut_p = jnp.pad(user_table.T, ((0, 0), (0, VP - V)))
    ot_p = jnp.pad(origin_table.T, ((0, 0), (0, VP - V)))
    u_sup, o_sup = _sc_gather(umod, dmod, ut_p, ot_p,
                              _tail_packed(user_table),
                              _tail_packed(origin_table))
    return _mlp(u_sup, o_sup, uq, oq, W1, b1, W2, b2, W3, b3)


# speedup vs baseline: 1.7984x; 1.7984x over previous
import functools

import jax
import jax.numpy as jnp
from jax import lax
from jax.experimental import pallas as pl
from jax.experimental.pallas import tpu as pltpu
from jax.experimental.pallas import tpu_sc as plsc

NC, NS = 2, 16
NW = NC * NS
CH = 128


def _sc_gather_one(idx, table):
    B = idx.shape[0]
    D = table.shape[1]
    b_per_w = B // NW
    n_ch = b_per_w // CH

    idx3 = idx.reshape(NW, n_ch, CH).astype(jnp.int32)

    mesh = plsc.VectorSubcoreMesh(core_axis_name="c", subcore_axis_name="s")

    @functools.partial(
        pl.kernel,
        out_type=jax.ShapeDtypeStruct((B, D), jnp.float32),
        mesh=mesh,
        scratch_types=[
            pltpu.VMEM((n_ch, CH), jnp.int32),
            pltpu.VMEM((b_per_w, D), jnp.float32),
            pltpu.SemaphoreType.DMA,
        ],
        compiler_params=pltpu.CompilerParams(use_tc_tiling_on_sc=False),
    )
    def gather_kernel(idx_hbm, tab_hbm, out_hbm, idx_v, rows_v, sem):
        wid = lax.axis_index("s") * NC + lax.axis_index("c")
        base = wid * b_per_w
        pltpu.sync_copy(idx_hbm.at[wid], idx_v)
        copies = []
        for j in range(n_ch):
            copies.append(pltpu.async_copy(
                tab_hbm.at[idx_v.at[j]], rows_v.at[pl.ds(j * CH, CH)], sem))
        for c in copies:
            c.wait()
        pltpu.sync_copy(rows_v, out_hbm.at[pl.ds(base, b_per_w)])

    return gather_kernel(idx3, table)


def _mlp_body(u_ref, o_ref, w1u_ref, w1o_ref, b1_ref, w2_ref, b2_ref,
              w3t_ref, b3_ref, out_ref):
    h1 = jnp.dot(u_ref[...], w1u_ref[...], preferred_element_type=jnp.float32)
    h1 = h1 + jnp.dot(o_ref[...], w1o_ref[...],
                      preferred_element_type=jnp.float32)
    h1 = jnp.maximum(h1 + b1_ref[...], 0.0)
    h2 = jnp.dot(h1, w2_ref[...], preferred_element_type=jnp.float32)
    h2 = jnp.maximum(h2 + b2_ref[...], 0.0)
    out_ref[...] = (jnp.sum(h2 * w3t_ref[...], axis=1, keepdims=True)
                    + b3_ref[...])


def _mlp(u_emb, o_emb, W1, b1, W2, b2, W3, b3, chunk=2048):
    B, D = u_emb.shape
    H1 = W1.shape[1]
    H2 = W2.shape[1]
    w1u = W1[:D]
    w1o = W1[D:]
    b1r = b1.reshape(1, H1)
    b2r = b2.reshape(1, H2)
    w3t = W3.reshape(1, H2)
    b3r = b3.reshape(1, 1)
    grid = (B // chunk,)
    return pl.pallas_call(
        _mlp_body,
        grid=grid,
        in_specs=[
            pl.BlockSpec((chunk, D), lambda i: (i, 0)),
            pl.BlockSpec((chunk, D), lambda i: (i, 0)),
            pl.BlockSpec((D, H1), lambda i: (0, 0)),
            pl.BlockSpec((D, H1), lambda i: (0, 0)),
            pl.BlockSpec((1, H1), lambda i: (0, 0)),
            pl.BlockSpec((H1, H2), lambda i: (0, 0)),
            pl.BlockSpec((1, H2), lambda i: (0, 0)),
            pl.BlockSpec((1, H2), lambda i: (0, 0)),
            pl.BlockSpec((1, 1), lambda i: (0, 0)),
        ],
        out_specs=pl.BlockSpec((chunk, 1), lambda i: (i, 0)),
        out_shape=jax.ShapeDtypeStruct((B, 1), jnp.float32),
    )(u_emb, o_emb, w1u, w1o, b1r, W2, b2r, w3t, b3r)


def kernel(user_id, destination, user_table, origin_table,
           W1, b1, W2, b2, W3, b3):
    u_emb = _sc_gather_one(user_id, user_table)
    o_emb = _sc_gather_one(destination, origin_table)
    return _mlp(u_emb, o_emb, W1, b1, W2, b2, W3, b3)
